# baseline (speedup 1.0000x reference)
import functools

import jax
import jax.numpy as jnp
from jax import lax
from jax.experimental import pallas as pl
from jax.experimental.pallas import tpu as pltpu
from jax.experimental.pallas import tpu_sc as plsc

N_ELECTRODES = 100000
EMBED_DIM = 128

NUM_WORKERS = 32
CHUNK = 128
CHUNKS_PER_W = 25
ROWS_PER_W = CHUNK * CHUNKS_PER_W
B_PAD = NUM_WORKERS * ROWS_PER_W

_mesh = plsc.VectorSubcoreMesh(core_axis_name="c", subcore_axis_name="s")


@functools.partial(
    pl.kernel,
    out_type=jax.ShapeDtypeStruct((B_PAD, EMBED_DIM), jnp.float32),
    mesh=_mesh,
    scratch_types=[
        pltpu.VMEM((CHUNKS_PER_W, CHUNK), jnp.int32),
        pltpu.VMEM((CHUNK, EMBED_DIM), jnp.float32),
        pltpu.VMEM((CHUNK, EMBED_DIM), jnp.float32),
        pltpu.SemaphoreType.DMA,
        pltpu.SemaphoreType.DMA,
        pltpu.SemaphoreType.DMA,
        pltpu.SemaphoreType.DMA,
    ],
)
def _gather_kernel(table_hbm, idx_hbm, out_hbm, idx_v, buf0, buf1,
                   gsem0, gsem1, wsem0, wsem1):
    wid = lax.axis_index("s") * 2 + lax.axis_index("c")
    base = wid * ROWS_PER_W

    pltpu.sync_copy(idx_hbm.at[wid], idx_v)

    bufs = (buf0, buf1)
    gsems = (gsem0, gsem1)
    wsems = (wsem0, wsem1)

    def gather_start(j, b):
        pltpu.async_copy(table_hbm.at[idx_v.at[j]], bufs[b], gsems[b])

    def gather_wait(b):
        pltpu.make_async_copy(table_hbm.at[idx_v.at[0]], bufs[b],
                              gsems[b]).wait()

    def write_start(j, b):
        pltpu.async_copy(bufs[b], out_hbm.at[pl.ds(base + j * CHUNK, CHUNK)],
                         wsems[b])

    def write_wait(b):
        pltpu.make_async_copy(bufs[b], out_hbm.at[pl.ds(base, CHUNK)],
                              wsems[b]).wait()

    gather_start(0, 0)
    gather_start(1, 1)

    def body(i, carry):
        j = i * 2
        gather_wait(0)
        write_start(j, 0)
        write_wait(0)
        gather_start(j + 2, 0)
        gather_wait(1)
        write_start(j + 1, 1)
        write_wait(1)

        @pl.when(j + 3 < CHUNKS_PER_W)
        def _():
            gather_start(j + 3, 1)

        return carry

    lax.fori_loop(0, (CHUNKS_PER_W - 1) // 2, body, 0)

    gather_wait(0)
    write_start(CHUNKS_PER_W - 1, 0)
    write_wait(0)


def kernel(table, permutation, subject_id=0):
    idx = permutation.astype(jnp.int32)
    idx_pad = jnp.pad(idx, (0, B_PAD - idx.shape[0]))
    idx3d = idx_pad.reshape(NUM_WORKERS, CHUNKS_PER_W, CHUNK)
    out = _gather_kernel(table, idx3d)
    return out[:N_ELECTRODES]
